# baseline (speedup 1.0000x reference)
import functools

import jax
import jax.numpy as jnp
from jax import lax
from jax.experimental import pallas as pl
from jax.experimental.pallas import tpu as pltpu
from jax.experimental.pallas import tpu_sc as plsc

L = 16
NC, NS = 2, 16
NW = NC * NS
HALVES = 2


@functools.lru_cache(maxsize=None)
def _make_sc_kernel(B, D):
    assert B % (8 * NW * HALVES) == 0 and D % L == 0
    BW = B // NW
    BH = BW // HALVES
    GROUPS = BH // L
    DV = D // L
    W = 2 * D

    mesh = plsc.VectorSubcoreMesh(
        core_axis_name="c", subcore_axis_name="s",
        num_cores=NC, num_subcores=NS)

    @functools.partial(
        pl.kernel,
        out_type=jax.ShapeDtypeStruct((B,), jnp.float32),
        mesh=mesh,
        scratch_types=[
            pltpu.VMEM((BW,), jnp.int32),
            pltpu.VMEM((BW,), jnp.int32),
            pltpu.VMEM((BW,), jnp.int32),
            pltpu.VMEM((BW,), jnp.int32),
            pltpu.VMEM((BW,), jnp.int32),
            pltpu.VMEM((BW,), jnp.int32),
            pltpu.VMEM((BH, W), jnp.float32),
            pltpu.VMEM((BH, W), jnp.float32),
            pltpu.VMEM((2, L, 128), jnp.float32),
            pltpu.VMEM((2, L, 128), jnp.float32),
            pltpu.VMEM((BW,), jnp.float32),
            pltpu.VMEM((BW,), jnp.float32),
            pltpu.VMEM((BW,), jnp.float32),
            pltpu.SemaphoreType.DMA,
        ],
    )
    def sc_kernel(crow_hbm, arow_hbm, cemb_hbm, aemb_hbm, cbias_hbm,
                  abias_hbm, out_hbm,
                  idx_c, idx_a, pid_c, pid_a, bid_c, bid_a, rows_c, rows_a,
                  brow_c, brow_a, b_c, b_a, out_v, sem):
        wid = lax.axis_index("s") * NC + lax.axis_index("c")
        base = wid * BW

        pltpu.sync_copy(crow_hbm.at[pl.ds(base, BW)], idx_c)
        pltpu.sync_copy(arow_hbm.at[pl.ds(base, BW)], idx_a)

        def rowids(g, carry):
            iv_c = idx_c[pl.ds(g * L, L)]
            iv_a = idx_a[pl.ds(g * L, L)]
            pid_c[pl.ds(g * L, L)] = iv_c >> 1
            pid_a[pl.ds(g * L, L)] = iv_a >> 1
            bid_c[pl.ds(g * L, L)] = iv_c >> 7
            bid_a[pl.ds(g * L, L)] = iv_a >> 7
            return carry

        lax.fori_loop(0, BW // L, rowids, 0)

        rows_iota = lax.iota(jnp.int32, L)
        bfly = [rows_iota ^ s for s in (8, 4, 2, 1)]

        def bias_issue(g, buf):
            pltpu.async_copy(
                cbias_hbm.at[bid_c.at[pl.ds(g * L, L)]], brow_c.at[buf], sem)
            pltpu.async_copy(
                abias_hbm.at[bid_a.at[pl.ds(g * L, L)]], brow_a.at[buf], sem)

        def bias_drain(buf):
            pltpu.make_async_copy(
                cbias_hbm.at[pl.ds(0, L)], brow_c.at[buf], sem).wait()
            pltpu.make_async_copy(
                abias_hbm.at[pl.ds(0, L)], brow_a.at[buf], sem).wait()

        def bias_extract(g, buf):
            iv_c = idx_c[pl.ds(g * L, L)]
            iv_a = idx_a[pl.ds(g * L, L)]
            vc = jnp.zeros((L,), jnp.float32)
            va = jnp.zeros((L,), jnp.float32)
            for r in range(L):
                c = iv_c[r]
                a = iv_a[r]
                bc = brow_c[buf, r, pl.ds(((c >> 4) & 7) * L, L)]
                ba = brow_a[buf, r, pl.ds(((a >> 4) & 7) * L, L)]
                bc = bc.at[jnp.broadcast_to(c & 15, (L,))].get(
                    mode="promise_in_bounds")
                ba = ba.at[jnp.broadcast_to(a & 15, (L,))].get(
                    mode="promise_in_bounds")
                vc = jnp.where(rows_iota == r, bc, vc)
                va = jnp.where(rows_iota == r, ba, va)
            b_c[pl.ds(g * L, L)] = vc
            b_a[pl.ds(g * L, L)] = va

        bias_issue(0, 0)

        def bias_pipe(g, carry):
            bias_issue(g, lax.rem(g, 2))
            bias_drain(lax.rem(g - 1, 2))
            bias_extract(g - 1, lax.rem(g - 1, 2))
            return carry

        lax.fori_loop(1, BW // L, bias_pipe, 0)
        bias_drain(lax.rem(BW // L - 1, 2))
        bias_extract(BW // L - 1, lax.rem(BW // L - 1, 2))

        for h in range(HALVES):
            hb = h * BH

            d1 = pltpu.async_copy(
                cemb_hbm.at[pid_c.at[pl.ds(hb, BH)]], rows_c, sem)
            d2 = pltpu.async_copy(
                aemb_hbm.at[pid_a.at[pl.ds(hb, BH)]], rows_a, sem)
            d1.wait()
            d2.wait()

            def group(g, carry):
                acc = jnp.zeros((L,), jnp.float32)
                iv_c = idx_c[pl.ds(hb + g * L, L)]
                iv_a = idx_a[pl.ds(hb + g * L, L)]
                for r in range(L):
                    row = g * L + r
                    co = (iv_c[r] & 1) * D
                    ao = (iv_a[r] & 1) * D
                    p = (rows_c[row, pl.ds(co, L)]
                         * rows_a[row, pl.ds(ao, L)])
                    for d in range(1, DV):
                        p = p + (rows_c[row, pl.ds(co + d * L, L)]
                                 * rows_a[row, pl.ds(ao + d * L, L)])
                    for perm in bfly:
                        p = p + p.at[perm].get(mode="promise_in_bounds")
                    acc = jnp.where(rows_iota == r, p, acc)
                x = (acc + b_c[pl.ds(hb + g * L, L)]
                     + b_a[pl.ds(hb + g * L, L)])
                out_v[pl.ds(hb + g * L, L)] = 1.0 / (1.0 + jnp.exp(-x))
                return carry

            lax.fori_loop(0, GROUPS, group, 0)

        pltpu.sync_copy(out_v, out_hbm.at[pl.ds(base, BW)])

    return sc_kernel


def kernel(customer_row, article_row, customer_embed, art_embed,
           customer_bias, article_bias):
    B = customer_row.shape[0]
    NCU, D = customer_embed.shape
    NA = art_embed.shape[0]
    cb_rows = -(-NCU // 128)
    ab_rows = -(-NA // 128)
    cb = jnp.pad(customer_bias, ((0, cb_rows * 128 - NCU), (0, 0)))
    ab = jnp.pad(article_bias, ((0, ab_rows * 128 - NA), (0, 0)))
    fn = _make_sc_kernel(B, D)
    out = fn(customer_row, article_row,
             customer_embed.reshape(NCU // 2, 2 * D),
             art_embed.reshape(NA // 2, 2 * D),
             cb.reshape(cb_rows, 128), ab.reshape(ab_rows, 128))
    return out.reshape(B, 1)


# speedup vs baseline: 1.4729x; 1.4729x over previous
import functools

import jax
import jax.numpy as jnp
from jax import lax
from jax.experimental import pallas as pl
from jax.experimental.pallas import tpu as pltpu
from jax.experimental.pallas import tpu_sc as plsc

L = 16
NC, NS = 2, 16
NW = NC * NS
HALVES = 2


@functools.lru_cache(maxsize=None)
def _make_sc_kernel(B, D):
    assert B % (8 * NW * HALVES) == 0 and D % L == 0
    BW = B // NW
    BH = BW // HALVES
    GROUPS = BH // L
    DV = D // L
    W = 2 * D

    mesh = plsc.VectorSubcoreMesh(
        core_axis_name="c", subcore_axis_name="s",
        num_cores=NC, num_subcores=NS)

    @functools.partial(
        pl.kernel,
        out_type=jax.ShapeDtypeStruct((B,), jnp.float32),
        mesh=mesh,
        scratch_types=[
            pltpu.VMEM((BW,), jnp.int32),
            pltpu.VMEM((BW,), jnp.int32),
            pltpu.VMEM((BW,), jnp.int32),
            pltpu.VMEM((BW,), jnp.int32),
            pltpu.VMEM((BH, D), jnp.float32),
            pltpu.VMEM((BH, D), jnp.float32),
            pltpu.VMEM((2, L, 128), jnp.float32),
            pltpu.VMEM((2, L, 128), jnp.float32),
            pltpu.VMEM((BW,), jnp.float32),
            pltpu.VMEM((BW,), jnp.float32),
            pltpu.VMEM((BW,), jnp.float32),
            pltpu.SemaphoreType.DMA,
        ],
    )
    def sc_kernel(crow_hbm, arow_hbm, cemb_hbm, aemb_hbm, cbias_hbm,
                  abias_hbm, out_hbm,
                  idx_c, idx_a, bid_c, bid_a, rows_c, rows_a,
                  brow_c, brow_a, b_c, b_a, out_v, sem):
        wid = lax.axis_index("s") * NC + lax.axis_index("c")
        base = wid * BW

        pltpu.sync_copy(crow_hbm.at[pl.ds(base, BW)], idx_c)
        pltpu.sync_copy(arow_hbm.at[pl.ds(base, BW)], idx_a)

        def rowids(g, carry):
            iv_c = idx_c[pl.ds(g * L, L)]
            iv_a = idx_a[pl.ds(g * L, L)]
            bid_c[pl.ds(g * L, L)] = iv_c >> 7
            bid_a[pl.ds(g * L, L)] = iv_a >> 7
            return carry

        lax.fori_loop(0, BW // L, rowids, 0)

        rows_iota = lax.iota(jnp.int32, L)
        bfly = [rows_iota ^ s for s in (8, 4, 2, 1)]

        def bias_issue(g, buf):
            pltpu.async_copy(
                cbias_hbm.at[bid_c.at[pl.ds(g * L, L)]], brow_c.at[buf], sem)
            pltpu.async_copy(
                abias_hbm.at[bid_a.at[pl.ds(g * L, L)]], brow_a.at[buf], sem)

        def bias_drain(buf):
            pltpu.make_async_copy(
                cbias_hbm.at[pl.ds(0, L)], brow_c.at[buf], sem).wait()
            pltpu.make_async_copy(
                abias_hbm.at[pl.ds(0, L)], brow_a.at[buf], sem).wait()

        def bias_extract(g, buf):
            iv_c = idx_c[pl.ds(g * L, L)]
            iv_a = idx_a[pl.ds(g * L, L)]
            vc = jnp.zeros((L,), jnp.float32)
            va = jnp.zeros((L,), jnp.float32)
            for r in range(L):
                c = iv_c[r]
                a = iv_a[r]
                bc = brow_c[buf, r, pl.ds(((c >> 4) & 7) * L, L)]
                ba = brow_a[buf, r, pl.ds(((a >> 4) & 7) * L, L)]
                bc = bc.at[jnp.broadcast_to(c & 15, (L,))].get(
                    mode="promise_in_bounds")
                ba = ba.at[jnp.broadcast_to(a & 15, (L,))].get(
                    mode="promise_in_bounds")
                vc = jnp.where(rows_iota == r, bc, vc)
                va = jnp.where(rows_iota == r, ba, va)
            b_c[pl.ds(g * L, L)] = vc
            b_a[pl.ds(g * L, L)] = va

        bias_issue(0, 0)

        def bias_pipe(g, carry):
            bias_issue(g, lax.rem(g, 2))
            bias_drain(lax.rem(g - 1, 2))
            bias_extract(g - 1, lax.rem(g - 1, 2))
            return carry

        lax.fori_loop(1, BW // L, bias_pipe, 0)
        bias_drain(lax.rem(BW // L - 1, 2))
        bias_extract(BW // L - 1, lax.rem(BW // L - 1, 2))

        for h in range(HALVES):
            hb = h * BH

            def issue(g, carry):
                iv_c = idx_c[pl.ds(hb + g * L, L)]
                iv_a = idx_a[pl.ds(hb + g * L, L)]
                for r in range(L):
                    row = g * L + r
                    pltpu.async_copy(cemb_hbm.at[pl.ds(iv_c[r], 1), :],
                                     rows_c.at[pl.ds(row, 1), :], sem)
                    pltpu.async_copy(aemb_hbm.at[pl.ds(iv_a[r], 1), :],
                                     rows_a.at[pl.ds(row, 1), :], sem)
                return carry

            lax.fori_loop(0, GROUPS, issue, 0)

            pltpu.make_async_copy(
                cemb_hbm.at[pl.ds(0, BH), :], rows_c, sem).wait()
            pltpu.make_async_copy(
                aemb_hbm.at[pl.ds(0, BH), :], rows_a, sem).wait()

            def group(g, carry):
                acc = jnp.zeros((L,), jnp.float32)
                for r in range(L):
                    row = g * L + r
                    p = rows_c[row, pl.ds(0, L)] * rows_a[row, pl.ds(0, L)]
                    for d in range(1, DV):
                        p = p + (rows_c[row, pl.ds(d * L, L)]
                                 * rows_a[row, pl.ds(d * L, L)])
                    for perm in bfly:
                        p = p + p.at[perm].get(mode="promise_in_bounds")
                    acc = jnp.where(rows_iota == r, p, acc)
                x = (acc + b_c[pl.ds(hb + g * L, L)]
                     + b_a[pl.ds(hb + g * L, L)])
                out_v[pl.ds(hb + g * L, L)] = 1.0 / (1.0 + jnp.exp(-x))
                return carry

            lax.fori_loop(0, GROUPS, group, 0)

        pltpu.sync_copy(out_v, out_hbm.at[pl.ds(base, BW)])

    return sc_kernel


def kernel(customer_row, article_row, customer_embed, art_embed,
           customer_bias, article_bias):
    B = customer_row.shape[0]
    NCU, D = customer_embed.shape
    NA = art_embed.shape[0]
    cb_rows = -(-NCU // 128)
    ab_rows = -(-NA // 128)
    cb = jnp.pad(customer_bias, ((0, cb_rows * 128 - NCU), (0, 0)))
    ab = jnp.pad(article_bias, ((0, ab_rows * 128 - NA), (0, 0)))
    fn = _make_sc_kernel(B, D)
    out = fn(customer_row, article_row, customer_embed, art_embed,
             cb.reshape(cb_rows, 128), ab.reshape(ab_rows, 128))
    return out.reshape(B, 1)


# speedup vs baseline: 1.4755x; 1.0018x over previous
import functools

import jax
import jax.numpy as jnp
from jax import lax
from jax.experimental import pallas as pl
from jax.experimental.pallas import tpu as pltpu
from jax.experimental.pallas import tpu_sc as plsc

L = 16
NC, NS = 2, 16
NW = NC * NS
HALVES = 2


@functools.lru_cache(maxsize=None)
def _make_sc_kernel(B, D):
    assert B % (8 * NW * HALVES) == 0 and D % L == 0
    BW = B // NW
    BH = BW // HALVES
    GROUPS = BH // L
    DV = D // L
    W = 2 * D

    mesh = plsc.VectorSubcoreMesh(
        core_axis_name="c", subcore_axis_name="s",
        num_cores=NC, num_subcores=NS)

    @functools.partial(
        pl.kernel,
        out_type=jax.ShapeDtypeStruct((B,), jnp.float32),
        mesh=mesh,
        scratch_types=[
            pltpu.VMEM((BW,), jnp.int32),
            pltpu.VMEM((BW,), jnp.int32),
            pltpu.VMEM((BW,), jnp.int32),
            pltpu.VMEM((BW,), jnp.int32),
            pltpu.VMEM((BH, D), jnp.float32),
            pltpu.VMEM((BH, D), jnp.float32),
            pltpu.VMEM((2, L, 128), jnp.float32),
            pltpu.VMEM((2, L, 128), jnp.float32),
            pltpu.VMEM((BW,), jnp.float32),
            pltpu.VMEM((BW,), jnp.float32),
            pltpu.VMEM((BW,), jnp.float32),
            pltpu.SemaphoreType.DMA,
            pltpu.SemaphoreType.DMA,
        ],
    )
    def sc_kernel(crow_hbm, arow_hbm, cemb_hbm, aemb_hbm, cbias_hbm,
                  abias_hbm, out_hbm,
                  idx_c, idx_a, bid_c, bid_a, rows_c, rows_a,
                  brow_c, brow_a, b_c, b_a, out_v, sem, sem_b):
        wid = lax.axis_index("s") * NC + lax.axis_index("c")
        base = wid * BW

        pltpu.sync_copy(crow_hbm.at[pl.ds(base, BW)], idx_c)
        pltpu.sync_copy(arow_hbm.at[pl.ds(base, BW)], idx_a)

        def rowids(g, carry):
            iv_c = idx_c[pl.ds(g * L, L)]
            iv_a = idx_a[pl.ds(g * L, L)]
            bid_c[pl.ds(g * L, L)] = iv_c >> 7
            bid_a[pl.ds(g * L, L)] = iv_a >> 7
            return carry

        lax.fori_loop(0, BW // L, rowids, 0)

        rows_iota = lax.iota(jnp.int32, L)
        bfly = [rows_iota ^ s for s in (8, 4, 2, 1)]

        def bias_issue(g, buf):
            pltpu.async_copy(
                cbias_hbm.at[bid_c.at[pl.ds(g * L, L)]], brow_c.at[buf],
                sem_b)
            pltpu.async_copy(
                abias_hbm.at[bid_a.at[pl.ds(g * L, L)]], brow_a.at[buf],
                sem_b)

        def bias_drain(buf):
            pltpu.make_async_copy(
                cbias_hbm.at[pl.ds(0, L)], brow_c.at[buf], sem_b).wait()
            pltpu.make_async_copy(
                abias_hbm.at[pl.ds(0, L)], brow_a.at[buf], sem_b).wait()

        def bias_extract(g, buf):
            iv_c = idx_c[pl.ds(g * L, L)]
            iv_a = idx_a[pl.ds(g * L, L)]
            vc = jnp.zeros((L,), jnp.float32)
            va = jnp.zeros((L,), jnp.float32)
            for r in range(L):
                c = iv_c[r]
                a = iv_a[r]
                bc = brow_c[buf, r, pl.ds(((c >> 4) & 7) * L, L)]
                ba = brow_a[buf, r, pl.ds(((a >> 4) & 7) * L, L)]
                bc = bc.at[jnp.broadcast_to(c & 15, (L,))].get(
                    mode="promise_in_bounds")
                ba = ba.at[jnp.broadcast_to(a & 15, (L,))].get(
                    mode="promise_in_bounds")
                vc = jnp.where(rows_iota == r, bc, vc)
                va = jnp.where(rows_iota == r, ba, va)
            b_c[pl.ds(g * L, L)] = vc
            b_a[pl.ds(g * L, L)] = va

        def issue_half(hb):
            def issue(g, carry):
                iv_c = idx_c[pl.ds(hb + g * L, L)]
                iv_a = idx_a[pl.ds(hb + g * L, L)]
                for r in range(L):
                    row = g * L + r
                    pltpu.async_copy(cemb_hbm.at[pl.ds(iv_c[r], 1), :],
                                     rows_c.at[pl.ds(row, 1), :], sem)
                    pltpu.async_copy(aemb_hbm.at[pl.ds(iv_a[r], 1), :],
                                     rows_a.at[pl.ds(row, 1), :], sem)
                return carry

            lax.fori_loop(0, GROUPS, issue, 0)

        def drain_half():
            pltpu.make_async_copy(
                cemb_hbm.at[pl.ds(0, BH), :], rows_c, sem).wait()
            pltpu.make_async_copy(
                aemb_hbm.at[pl.ds(0, BH), :], rows_a, sem).wait()

        issue_half(0)

        bias_issue(0, 0)

        def bias_pipe(g, carry):
            bias_issue(g, lax.rem(g, 2))
            bias_drain(lax.rem(g - 1, 2))
            bias_extract(g - 1, lax.rem(g - 1, 2))
            return carry

        lax.fori_loop(1, BW // L, bias_pipe, 0)
        bias_drain(lax.rem(BW // L - 1, 2))
        bias_extract(BW // L - 1, lax.rem(BW // L - 1, 2))

        for h in range(HALVES):
            hb = h * BH
            if h > 0:
                issue_half(hb)
            drain_half()

            def group(g, carry):
                acc = jnp.zeros((L,), jnp.float32)
                for r in range(L):
                    row = g * L + r
                    p = rows_c[row, pl.ds(0, L)] * rows_a[row, pl.ds(0, L)]
                    for d in range(1, DV):
                        p = p + (rows_c[row, pl.ds(d * L, L)]
                                 * rows_a[row, pl.ds(d * L, L)])
                    for perm in bfly:
                        p = p + p.at[perm].get(mode="promise_in_bounds")
                    acc = jnp.where(rows_iota == r, p, acc)
                x = (acc + b_c[pl.ds(hb + g * L, L)]
                     + b_a[pl.ds(hb + g * L, L)])
                out_v[pl.ds(hb + g * L, L)] = 1.0 / (1.0 + jnp.exp(-x))
                return carry

            lax.fori_loop(0, GROUPS, group, 0)

        pltpu.sync_copy(out_v, out_hbm.at[pl.ds(base, BW)])

    return sc_kernel


def kernel(customer_row, article_row, customer_embed, art_embed,
           customer_bias, article_bias):
    B = customer_row.shape[0]
    NCU, D = customer_embed.shape
    NA = art_embed.shape[0]
    cb_rows = -(-NCU // 128)
    ab_rows = -(-NA // 128)
    cb = jnp.pad(customer_bias, ((0, cb_rows * 128 - NCU), (0, 0)))
    ab = jnp.pad(article_bias, ((0, ab_rows * 128 - NA), (0, 0)))
    fn = _make_sc_kernel(B, D)
    out = fn(customer_row, article_row, customer_embed, art_embed,
             cb.reshape(cb_rows, 128), ab.reshape(ab_rows, 128))
    return out.reshape(B, 1)


# speedup vs baseline: 1.4762x; 1.0004x over previous
import functools

import jax
import jax.numpy as jnp
from jax import lax
from jax.experimental import pallas as pl
from jax.experimental.pallas import tpu as pltpu
from jax.experimental.pallas import tpu_sc as plsc

L = 16
NC, NS = 2, 16
NW = NC * NS
HALVES = 2


@functools.lru_cache(maxsize=None)
def _make_sc_kernel(B, D):
    assert B % (8 * NW * HALVES) == 0 and D % L == 0
    BW = B // NW
    BH = BW // HALVES
    GROUPS = BH // L
    DV = D // L

    mesh = plsc.VectorSubcoreMesh(
        core_axis_name="c", subcore_axis_name="s",
        num_cores=NC, num_subcores=NS)

    @functools.partial(
        pl.kernel,
        out_type=jax.ShapeDtypeStruct((B,), jnp.float32),
        mesh=mesh,
        scratch_types=[
            pltpu.VMEM((BW,), jnp.int32),
            pltpu.VMEM((BW,), jnp.int32),
            pltpu.VMEM((BW,), jnp.int32),
            pltpu.VMEM((BW,), jnp.int32),
            pltpu.VMEM((BH, D), jnp.float32),
            pltpu.VMEM((BH, D), jnp.float32),
            pltpu.VMEM((2, L, 128), jnp.float32),
            pltpu.VMEM((2, L, 128), jnp.float32),
            pltpu.VMEM((BW,), jnp.float32),
            pltpu.VMEM((BW,), jnp.float32),
            pltpu.VMEM((BW,), jnp.float32),
            pltpu.SemaphoreType.DMA,
            pltpu.SemaphoreType.DMA,
        ],
    )
    def sc_kernel(crow_hbm, arow_hbm, cemb_hbm, aemb_hbm, cbias_hbm,
                  abias_hbm, out_hbm,
                  idx_c, idx_a, bid_c, bid_a, rows_c, rows_a,
                  brow_c, brow_a, b_c, b_a, out_v, sem, sem_b):
        wid = lax.axis_index("s") * NC + lax.axis_index("c")
        base = wid * BW

        pltpu.sync_copy(crow_hbm.at[pl.ds(base, BW)], idx_c)
        pltpu.sync_copy(arow_hbm.at[pl.ds(base, BW)], idx_a)

        def rowids(g, carry):
            iv_c = idx_c[pl.ds(g * L, L)]
            iv_a = idx_a[pl.ds(g * L, L)]
            bid_c[pl.ds(g * L, L)] = iv_c >> 7
            bid_a[pl.ds(g * L, L)] = iv_a >> 7
            return carry

        lax.fori_loop(0, BW // L, rowids, 0)

        rows_iota = lax.iota(jnp.int32, L)
        bfly = [rows_iota ^ s for s in (8, 4, 2, 1)]

        def bias_issue(g, buf):
            pltpu.async_copy(
                cbias_hbm.at[bid_c.at[pl.ds(g * L, L)]], brow_c.at[buf],
                sem_b)
            pltpu.async_copy(
                abias_hbm.at[bid_a.at[pl.ds(g * L, L)]], brow_a.at[buf],
                sem_b)

        def bias_drain(buf):
            pltpu.make_async_copy(
                cbias_hbm.at[pl.ds(0, L)], brow_c.at[buf], sem_b).wait()
            pltpu.make_async_copy(
                abias_hbm.at[pl.ds(0, L)], brow_a.at[buf], sem_b).wait()

        def bias_extract(g, buf):
            iv_c = idx_c[pl.ds(g * L, L)]
            iv_a = idx_a[pl.ds(g * L, L)]
            vc = jnp.zeros((L,), jnp.float32)
            va = jnp.zeros((L,), jnp.float32)
            for r in range(L):
                c = iv_c[r]
                a = iv_a[r]
                bc = brow_c[buf, r, pl.ds(((c >> 4) & 7) * L, L)]
                ba = brow_a[buf, r, pl.ds(((a >> 4) & 7) * L, L)]
                bc = bc.at[jnp.broadcast_to(c & 15, (L,))].get(
                    mode="promise_in_bounds")
                ba = ba.at[jnp.broadcast_to(a & 15, (L,))].get(
                    mode="promise_in_bounds")
                vc = jnp.where(rows_iota == r, bc, vc)
                va = jnp.where(rows_iota == r, ba, va)
            b_c[pl.ds(g * L, L)] = vc
            b_a[pl.ds(g * L, L)] = va

        def issue_half(hb):
            def issue(g, carry):
                iv_c = idx_c[pl.ds(hb + g * L, L)]
                iv_a = idx_a[pl.ds(hb + g * L, L)]
                for r in range(L):
                    row = g * L + r
                    pltpu.async_copy(cemb_hbm.at[pl.ds(iv_c[r], 1), :],
                                     rows_c.at[pl.ds(row, 1), :], sem)
                    pltpu.async_copy(aemb_hbm.at[pl.ds(iv_a[r], 1), :],
                                     rows_a.at[pl.ds(row, 1), :], sem)
                return carry

            lax.fori_loop(0, GROUPS, issue, 0)

        def drain_half():
            pltpu.make_async_copy(
                cemb_hbm.at[pl.ds(0, BH), :], rows_c, sem).wait()
            pltpu.make_async_copy(
                aemb_hbm.at[pl.ds(0, BH), :], rows_a, sem).wait()

        issue_half(0)

        bias_issue(0, 0)

        def bias_pipe(g, carry):
            bias_issue(g, lax.rem(g, 2))
            bias_drain(lax.rem(g - 1, 2))
            bias_extract(g - 1, lax.rem(g - 1, 2))
            return carry

        lax.fori_loop(1, BW // L, bias_pipe, 0)
        bias_drain(lax.rem(BW // L - 1, 2))
        bias_extract(BW // L - 1, lax.rem(BW // L - 1, 2))

        for h in range(HALVES):
            hb = h * BH
            if h > 0:
                issue_half(hb)
            drain_half()

            def group(g, carry):
                acc = jnp.zeros((L,), jnp.float32)
                for r in range(L):
                    row = g * L + r
                    p = rows_c[row, pl.ds(0, L)] * rows_a[row, pl.ds(0, L)]
                    for d in range(1, DV):
                        p = p + (rows_c[row, pl.ds(d * L, L)]
                                 * rows_a[row, pl.ds(d * L, L)])
                    for perm in bfly:
                        p = p + p.at[perm].get(mode="promise_in_bounds")
                    acc = jnp.where(rows_iota == r, p, acc)
                x = (acc + b_c[pl.ds(hb + g * L, L)]
                     + b_a[pl.ds(hb + g * L, L)])
                out_v[pl.ds(hb + g * L, L)] = 1.0 / (1.0 + jnp.exp(-x))
                return carry

            lax.fori_loop(0, GROUPS, group, 0)

        pltpu.sync_copy(out_v, out_hbm.at[pl.ds(base, BW)])

    return sc_kernel


def kernel(customer_row, article_row, customer_embed, art_embed,
           customer_bias, article_bias):
    B = customer_row.shape[0]
    NCU, D = customer_embed.shape
    NA = art_embed.shape[0]
    cb_rows = -(-NCU // 128)
    ab_rows = -(-NA // 128)
    cb = jnp.pad(customer_bias, ((0, cb_rows * 128 - NCU), (0, 0)))
    ab = jnp.pad(article_bias, ((0, ab_rows * 128 - NA), (0, 0)))
    fn = _make_sc_kernel(B, D)
    out = fn(customer_row, article_row, customer_embed, art_embed,
             cb.reshape(cb_rows, 128), ab.reshape(ab_rows, 128))
    return out.reshape(B, 1)
